# baseline (speedup 1.0000x reference)
import functools

import jax
import jax.numpy as jnp
from jax import lax
from jax.experimental import pallas as pl
from jax.experimental.pallas import tpu as pltpu
from jax.experimental.pallas import tpu_sc as plsc

B, S, H = 2, 2048, 2048
E, TOPK, DFF = 8, 2, 8192
AUX_COEF = 0.001
Z_COEF = 0.001

T = B * S
NA = TOPK * T
TILE = 512
LOG2_TILE = 9
NPAD = 11776
NBLK = NPAD // TILE
FT = 1024
NF = DFF // FT
RB = 128
NRB = T // RB
NW = 32
SPAN = NA // NW
TSPAN = T // NW

_mesh = plsc.VectorSubcoreMesh(core_axis_name="c", subcore_axis_name="s")


def _router_body(x_ref, wr_ref, ep_ref, cnt_ref, loss_ref, acc_ref):
    i = pl.program_id(0)
    xb = x_ref[...].astype(jnp.bfloat16)
    wr = wr_ref[...].astype(jnp.bfloat16)
    logits = lax.dot_general(xb, wr, (((1,), (1,)), ((), ())),
                             preferred_element_type=jnp.float32)
    m = jnp.max(logits, axis=1, keepdims=True)
    p = jnp.exp(logits - m)
    probs = p / jnp.sum(p, axis=1, keepdims=True)
    idx = lax.broadcasted_iota(jnp.int32, (RB, E), 1)
    p1 = jnp.max(probs, axis=1, keepdims=True)
    top1 = jnp.min(jnp.where(probs == p1, idx, E), axis=1, keepdims=True)
    pm = jnp.where(idx == top1, -jnp.inf, probs)
    p2 = jnp.max(pm, axis=1, keepdims=True)
    top2 = jnp.min(jnp.where(pm == p2, idx, E), axis=1, keepdims=True)
    ep_ref[:, 0:1] = top1
    ep_ref[:, 1:2] = top2
    idx16 = lax.broadcasted_iota(jnp.int32, (RB, 16), 1)
    cnt = (jnp.sum((idx16 == top1).astype(jnp.int32), axis=0, keepdims=True)
           + jnp.sum((idx16 == top2).astype(jnp.int32), axis=0, keepdims=True))
    cnt_ref[0] = cnt
    lse = m + jnp.log(jnp.sum(p, axis=1, keepdims=True))
    aux_p = jnp.sum(lse) - jnp.sum(logits) / E
    z_p = jnp.sum(logits * logits)

    @pl.when(i == 0)
    def _():
        acc_ref[0] = 0.0
        acc_ref[1] = 0.0

    acc_ref[0] += aux_p
    acc_ref[1] += z_p

    @pl.when(i == NRB - 1)
    def _():
        aux = (acc_ref[0] - T * jnp.log(jnp.float32(E))) / B * AUX_COEF
        z = acc_ref[1] / (T * E) * Z_COEF
        loss_ref[...] = jnp.broadcast_to(aux + z, (1, 1))


def _router(x2d, W_router):
    return pl.pallas_call(
        _router_body,
        grid=(NRB,),
        in_specs=[pl.BlockSpec((RB, H), lambda i: (i, 0)),
                  pl.BlockSpec((E, H), lambda i: (0, 0))],
        out_specs=[pl.BlockSpec((RB, TOPK), lambda i: (i, 0)),
                   pl.BlockSpec((1, 1, 16), lambda i: (i, 0, 0)),
                   pl.BlockSpec((1, 1), lambda i: (0, 0))],
        out_shape=[jax.ShapeDtypeStruct((T, TOPK), jnp.int32),
                   jax.ShapeDtypeStruct((NRB, 1, 16), jnp.int32),
                   jax.ShapeDtypeStruct((1, 1), jnp.float32)],
        scratch_shapes=[pltpu.SMEM((2,), jnp.float32)],
    )(x2d, W_router)


@functools.partial(
    pl.kernel,
    out_type=[jax.ShapeDtypeStruct((NA,), jnp.int32),
              jax.ShapeDtypeStruct((32,), jnp.int32)],
    mesh=_mesh,
    scratch_types=[pltpu.VMEM((SPAN,), jnp.int32),
                   pltpu.VMEM((SPAN,), jnp.int32),
                   pltpu.VMEM((NW * 16,), jnp.int32),
                   pltpu.VMEM((32,), jnp.int32)],
    compiler_params=pltpu.CompilerParams(needs_layout_passes=False),
)
def _dispatch(e_hbm, cnt_hbm, slots_hbm, b2e_hbm, eb_v, sl_v, allc_v, b2e_v):
    cid = lax.axis_index("c")
    sid = lax.axis_index("s")
    wid = sid * 2 + cid
    base = wid * SPAN
    pltpu.sync_copy(cnt_hbm, allc_v)
    pltpu.sync_copy(e_hbm.at[pl.ds(base, SPAN)], eb_v)
    lane = lax.iota(jnp.int32, 16)
    tot = jnp.zeros((16,), jnp.int32)
    pre = jnp.zeros((16,), jnp.int32)
    for w in range(NW):
        cw = allc_v[pl.ds(w * 16, 16)]
        tot = tot + cw
        pre = pre + jnp.where(w < wid, cw, 0)
    ptot = ((tot + (TILE - 1)) >> LOG2_TILE) << LOG2_TILE
    cums = plsc.cumsum(ptot)
    offs = cums - ptot
    run = offs + pre
    for ch in range(SPAN // 16):
        ev = eb_v[pl.ds(ch * 16, 16)]
        sl = jnp.zeros((16,), jnp.int32)
        for v in range(E):
            msk = ev == v
            incl = plsc.cumsum(jnp.where(msk, 1, 0))
            basev = jnp.sum(jnp.where(lane == v, run, 0))
            sl = jnp.where(msk, basev + incl - 1, sl)
            c = plsc.all_reduce_population_count(msk)
            run = run + jnp.where(lane == v, c, 0)
        sl_v[pl.ds(ch * 16, 16)] = sl
    pltpu.sync_copy(sl_v, slots_hbm.at[pl.ds(base, SPAN)])

    @pl.when(wid == 0)
    def _():
        offs_blk = offs >> LOG2_TILE
        total_blk = jnp.sum(jnp.where(lane == E - 1, cums, 0)) >> LOG2_TILE
        for cc in range(2):
            jv = lane + cc * 16
            acc = jnp.full((16,), -1, jnp.int32)
            for v in range(E):
                ob = jnp.sum(jnp.where(lane == v, offs_blk, 0))
                acc = acc + jnp.where(jv >= ob, 1, 0)
            b2e_v[pl.ds(cc * 16, 16)] = jnp.where(jv < total_blk, acc, -1)
        pltpu.sync_copy(b2e_v, b2e_hbm)


@functools.partial(
    pl.kernel,
    out_type=jax.ShapeDtypeStruct((NPAD, H), jnp.float32),
    mesh=_mesh,
    scratch_types=[pltpu.VMEM((16,), jnp.int32),
                   pltpu.VMEM((16,), jnp.int32),
                   pltpu.VMEM((16, H), jnp.float32)],
)
def _scatter_rows(x_hbm, se_hbm, so_hbm, xs_hbm, ie_v, io_v, rows_v):
    cid = lax.axis_index("c")
    sid = lax.axis_index("s")
    wid = sid * 2 + cid
    t0 = wid * TSPAN

    def body(i, carry):
        tt = t0 + i * 16
        pltpu.sync_copy(se_hbm.at[pl.ds(tt, 16)], ie_v)
        pltpu.sync_copy(so_hbm.at[pl.ds(tt, 16)], io_v)
        pltpu.sync_copy(x_hbm.at[pl.ds(tt, 16)], rows_v)
        pltpu.sync_copy(rows_v, xs_hbm.at[ie_v])
        pltpu.sync_copy(rows_v, xs_hbm.at[io_v])
        return carry

    lax.fori_loop(0, TSPAN // 16, body, 0)


def _ffn_body(b2e_ref, x_ref, w1_ref, b1_ref, w2_ref, b2_ref, y_ref):
    j = pl.program_id(0)
    f = pl.program_id(1)
    fe = jnp.where((j & 1) == 1, NF - 1 - f, f)
    e = b2e_ref[j]

    @pl.when(e >= 0)
    def _():
        xb = x_ref[...].astype(jnp.bfloat16)
        w1 = w1_ref[0].astype(jnp.bfloat16)
        h = lax.dot_general(xb, w1, (((1,), (1,)), ((), ())),
                            preferred_element_type=jnp.float32)
        h = h + b1_ref[0]
        h = h * lax.logistic(h)
        hb = h.astype(jnp.bfloat16)
        w2 = w2_ref[0].astype(jnp.bfloat16)
        yk = lax.dot_general(hb, w2, (((1,), (1,)), ((), ())),
                             preferred_element_type=jnp.float32)

        @pl.when(f == 0)
        def _():
            y_ref[...] = yk + b2_ref[0]

        @pl.when(f != 0)
        def _():
            y_ref[...] = y_ref[...] + yk


def _ffn(b2e, xs, W1, b1, W2, b2):
    def _e(s, j):
        return jnp.maximum(s[j], 0)

    def _fe(j, f):
        return jnp.where((j & 1) == 1, NF - 1 - f, f)

    def _live(s, j, v, dead):
        return jnp.where(s[j] < 0, dead, v)

    grid_spec = pltpu.PrefetchScalarGridSpec(
        num_scalar_prefetch=1,
        grid=(NBLK, NF),
        in_specs=[
            pl.BlockSpec((TILE, H), lambda j, f, s: (_live(s, j, j, 0), 0)),
            pl.BlockSpec((1, FT, H),
                         lambda j, f, s: (_e(s, j), _live(s, j, _fe(j, f), 0), 0)),
            pl.BlockSpec((1, 1, FT),
                         lambda j, f, s: (
                             _live(s, j, s[j] * NF + _fe(j, f), 0), 0, 0)),
            pl.BlockSpec((1, H, FT),
                         lambda j, f, s: (_e(s, j), 0, _live(s, j, _fe(j, f), 0))),
            pl.BlockSpec((1, 1, H), lambda j, f, s: (_e(s, j), 0, 0)),
        ],
        out_specs=pl.BlockSpec((TILE, H), lambda j, f, s: (j, 0)),
    )
    return pl.pallas_call(
        _ffn_body,
        grid_spec=grid_spec,
        out_shape=jax.ShapeDtypeStruct((NPAD, H), jnp.float32),
        compiler_params=pltpu.CompilerParams(
            dimension_semantics=("arbitrary", "arbitrary")),
    )(b2e, xs, W1, b1.reshape(E * NF, 1, FT), W2, b2.reshape(E, 1, H))


@functools.partial(
    pl.kernel,
    out_type=[jax.ShapeDtypeStruct((T, H), jnp.float32),
              jax.ShapeDtypeStruct((T, H), jnp.float32)],
    mesh=_mesh,
    scratch_types=[pltpu.VMEM((16,), jnp.int32),
                   pltpu.VMEM((16,), jnp.int32),
                   pltpu.VMEM((16, H), jnp.float32),
                   pltpu.VMEM((16, H), jnp.float32),
                   pltpu.SemaphoreType.DMA,
                   pltpu.SemaphoreType.DMA],
)
def _combine(y_hbm, se_hbm, so_hbm, ya_hbm, yb_hbm,
             ie_v, io_v, ra_v, rb_v, sem_a, sem_b):
    cid = lax.axis_index("c")
    sid = lax.axis_index("s")
    wid = sid * 2 + cid
    t0 = wid * TSPAN

    def body(i, carry):
        tt = t0 + i * 16
        pltpu.sync_copy(se_hbm.at[pl.ds(tt, 16)], ie_v)
        pltpu.sync_copy(so_hbm.at[pl.ds(tt, 16)], io_v)
        ca = pltpu.async_copy(y_hbm.at[ie_v], ra_v, sem_a)
        cb = pltpu.async_copy(y_hbm.at[io_v], rb_v, sem_b)
        ca.wait()
        cb.wait()
        pltpu.sync_copy(ra_v, ya_hbm.at[pl.ds(tt, 16)])
        pltpu.sync_copy(rb_v, yb_hbm.at[pl.ds(tt, 16)])
        return carry

    lax.fori_loop(0, TSPAN // 16, body, 0)


def _add_body(a_ref, b_ref, o_ref):
    o_ref[...] = a_ref[...] + b_ref[...]


def _add(ya, yb):
    return pl.pallas_call(
        _add_body,
        grid=(T // 512,),
        in_specs=[pl.BlockSpec((512, H), lambda i: (i, 0)),
                  pl.BlockSpec((512, H), lambda i: (i, 0))],
        out_specs=pl.BlockSpec((512, H), lambda i: (i, 0)),
        out_shape=jax.ShapeDtypeStruct((T, H), jnp.float32),
    )(ya, yb)


def kernel(hidden_states, W_router, W1, b1, W2, b2):
    x2d = hidden_states.reshape(T, H)
    ep, cnts, loss11 = _router(x2d, W_router)
    e_flat = ep.reshape(NA)
    cnt_flat = cnts.reshape(NW * 16)
    slots, b2e = _dispatch(e_flat, cnt_flat)
    se = slots[0::2]
    so = slots[1::2]
    xs = _scatter_rows(x2d, se, so)
    y = _ffn(b2e, xs, W1, b1, W2, b2)
    ya, yb = _combine(y, se, so)
    out = _add(ya, yb).reshape(B, S, H)
    return (out, loss11[0, 0])


# speedup vs baseline: 1.0098x; 1.0098x over previous
import functools

import jax
import jax.numpy as jnp
from jax import lax
from jax.experimental import pallas as pl
from jax.experimental.pallas import tpu as pltpu
from jax.experimental.pallas import tpu_sc as plsc

B, S, H = 2, 2048, 2048
E, TOPK, DFF = 8, 2, 8192
AUX_COEF = 0.001
Z_COEF = 0.001

T = B * S
NA = TOPK * T
TILE = 512
LOG2_TILE = 9
NPAD = 11776
NBLK = NPAD // TILE
FT = 1024
NF = DFF // FT
RB = 128
NRB = T // RB
NW = 32
SPAN = NA // NW
TSPAN = T // NW

_mesh = plsc.VectorSubcoreMesh(core_axis_name="c", subcore_axis_name="s")


def _router_body(x_ref, wr_ref, ep_ref, cnt_ref, loss_ref, acc_ref):
    i = pl.program_id(0)
    xb = x_ref[...].astype(jnp.bfloat16)
    wr = wr_ref[...].astype(jnp.bfloat16)
    logits = lax.dot_general(xb, wr, (((1,), (1,)), ((), ())),
                             preferred_element_type=jnp.float32)
    m = jnp.max(logits, axis=1, keepdims=True)
    p = jnp.exp(logits - m)
    probs = p / jnp.sum(p, axis=1, keepdims=True)
    idx = lax.broadcasted_iota(jnp.int32, (RB, E), 1)
    p1 = jnp.max(probs, axis=1, keepdims=True)
    top1 = jnp.min(jnp.where(probs == p1, idx, E), axis=1, keepdims=True)
    pm = jnp.where(idx == top1, -jnp.inf, probs)
    p2 = jnp.max(pm, axis=1, keepdims=True)
    top2 = jnp.min(jnp.where(pm == p2, idx, E), axis=1, keepdims=True)
    ep_ref[:, 0:1] = top1
    ep_ref[:, 1:2] = top2
    idx16 = lax.broadcasted_iota(jnp.int32, (RB, 16), 1)
    cnt = (jnp.sum((idx16 == top1).astype(jnp.int32), axis=0, keepdims=True)
           + jnp.sum((idx16 == top2).astype(jnp.int32), axis=0, keepdims=True))
    cnt_ref[0] = cnt
    lse = m + jnp.log(jnp.sum(p, axis=1, keepdims=True))
    aux_p = jnp.sum(lse) - jnp.sum(logits) / E
    z_p = jnp.sum(logits * logits)

    @pl.when(i == 0)
    def _():
        acc_ref[0] = 0.0
        acc_ref[1] = 0.0

    acc_ref[0] += aux_p
    acc_ref[1] += z_p

    @pl.when(i == NRB - 1)
    def _():
        aux = (acc_ref[0] - T * jnp.log(jnp.float32(E))) / B * AUX_COEF
        z = acc_ref[1] / (T * E) * Z_COEF
        loss_ref[...] = jnp.broadcast_to(aux + z, (1, 1))


def _router(x2d, W_router):
    return pl.pallas_call(
        _router_body,
        grid=(NRB,),
        in_specs=[pl.BlockSpec((RB, H), lambda i: (i, 0)),
                  pl.BlockSpec((E, H), lambda i: (0, 0))],
        out_specs=[pl.BlockSpec((RB, TOPK), lambda i: (i, 0)),
                   pl.BlockSpec((1, 1, 16), lambda i: (i, 0, 0)),
                   pl.BlockSpec((1, 1), lambda i: (0, 0))],
        out_shape=[jax.ShapeDtypeStruct((T, TOPK), jnp.int32),
                   jax.ShapeDtypeStruct((NRB, 1, 16), jnp.int32),
                   jax.ShapeDtypeStruct((1, 1), jnp.float32)],
        scratch_shapes=[pltpu.SMEM((2,), jnp.float32)],
    )(x2d, W_router)


@functools.partial(
    pl.kernel,
    out_type=[jax.ShapeDtypeStruct((NA,), jnp.int32),
              jax.ShapeDtypeStruct((32,), jnp.int32)],
    mesh=_mesh,
    scratch_types=[pltpu.VMEM((SPAN,), jnp.int32),
                   pltpu.VMEM((SPAN,), jnp.int32),
                   pltpu.VMEM((NW * 16,), jnp.int32),
                   pltpu.VMEM((32,), jnp.int32)],
    compiler_params=pltpu.CompilerParams(needs_layout_passes=False),
)
def _dispatch(e_hbm, cnt_hbm, slots_hbm, b2e_hbm, eb_v, sl_v, allc_v, b2e_v):
    cid = lax.axis_index("c")
    sid = lax.axis_index("s")
    wid = sid * 2 + cid
    base = wid * SPAN
    pltpu.sync_copy(cnt_hbm, allc_v)
    pltpu.sync_copy(e_hbm.at[pl.ds(base, SPAN)], eb_v)
    lane = lax.iota(jnp.int32, 16)
    tot = jnp.zeros((16,), jnp.int32)
    pre = jnp.zeros((16,), jnp.int32)
    for w in range(NW):
        cw = allc_v[pl.ds(w * 16, 16)]
        tot = tot + cw
        pre = pre + jnp.where(w < wid, cw, 0)
    ptot = ((tot + (TILE - 1)) >> LOG2_TILE) << LOG2_TILE
    cums = plsc.cumsum(ptot)
    offs = cums - ptot
    run = offs + pre
    for ch in range(SPAN // 16):
        ev = eb_v[pl.ds(ch * 16, 16)]
        sl = jnp.zeros((16,), jnp.int32)
        for v in range(E):
            msk = ev == v
            incl = plsc.cumsum(jnp.where(msk, 1, 0))
            basev = jnp.sum(jnp.where(lane == v, run, 0))
            sl = jnp.where(msk, basev + incl - 1, sl)
            c = plsc.all_reduce_population_count(msk)
            run = run + jnp.where(lane == v, c, 0)
        sl_v[pl.ds(ch * 16, 16)] = sl
    pltpu.sync_copy(sl_v, slots_hbm.at[pl.ds(base, SPAN)])

    @pl.when(wid == 0)
    def _():
        offs_blk = offs >> LOG2_TILE
        total_blk = jnp.sum(jnp.where(lane == E - 1, cums, 0)) >> LOG2_TILE
        for cc in range(2):
            jv = lane + cc * 16
            acc = jnp.full((16,), -1, jnp.int32)
            for v in range(E):
                ob = jnp.sum(jnp.where(lane == v, offs_blk, 0))
                acc = acc + jnp.where(jv >= ob, 1, 0)
            b2e_v[pl.ds(cc * 16, 16)] = jnp.where(jv < total_blk, acc, -1)
        pltpu.sync_copy(b2e_v, b2e_hbm)


@functools.partial(
    pl.kernel,
    out_type=jax.ShapeDtypeStruct((NPAD, H), jnp.float32),
    mesh=_mesh,
    scratch_types=[pltpu.VMEM((16,), jnp.int32),
                   pltpu.VMEM((16,), jnp.int32),
                   pltpu.VMEM((16, H), jnp.float32)],
)
def _scatter_rows(x_hbm, se_hbm, so_hbm, xs_hbm, ie_v, io_v, rows_v):
    cid = lax.axis_index("c")
    sid = lax.axis_index("s")
    wid = sid * 2 + cid
    t0 = wid * TSPAN

    def body(i, carry):
        tt = t0 + i * 16
        pltpu.sync_copy(se_hbm.at[pl.ds(tt, 16)], ie_v)
        pltpu.sync_copy(so_hbm.at[pl.ds(tt, 16)], io_v)
        pltpu.sync_copy(x_hbm.at[pl.ds(tt, 16)], rows_v)
        pltpu.sync_copy(rows_v, xs_hbm.at[ie_v])
        pltpu.sync_copy(rows_v, xs_hbm.at[io_v])
        return carry

    lax.fori_loop(0, TSPAN // 16, body, 0)


def _ffn_body(b2e_ref, x_ref, w1_ref, b1_ref, w2_ref, b2_ref, y_ref):
    j = pl.program_id(0)
    f = pl.program_id(1)
    fe = jnp.where((j & 1) == 1, NF - 1 - f, f)
    e = b2e_ref[j]

    @pl.when(e >= 0)
    def _():
        h = lax.dot_general(x_ref[...], w1_ref[0], (((1,), (1,)), ((), ())),
                            precision=lax.Precision.DEFAULT,
                            preferred_element_type=jnp.float32)
        h = h + b1_ref[0]
        h = h * lax.logistic(h)
        yk = lax.dot_general(h, w2_ref[0], (((1,), (1,)), ((), ())),
                             precision=lax.Precision.DEFAULT,
                             preferred_element_type=jnp.float32)

        @pl.when(f == 0)
        def _():
            y_ref[...] = yk + b2_ref[0]

        @pl.when(f != 0)
        def _():
            y_ref[...] = y_ref[...] + yk


def _ffn(b2e, xs, W1, b1, W2, b2):
    def _e(s, j):
        return jnp.maximum(s[j], 0)

    def _fe(j, f):
        return jnp.where((j & 1) == 1, NF - 1 - f, f)

    def _live(s, j, v, dead):
        return jnp.where(s[j] < 0, dead, v)

    grid_spec = pltpu.PrefetchScalarGridSpec(
        num_scalar_prefetch=1,
        grid=(NBLK, NF),
        in_specs=[
            pl.BlockSpec((TILE, H), lambda j, f, s: (_live(s, j, j, 0), 0)),
            pl.BlockSpec((1, FT, H),
                         lambda j, f, s: (_e(s, j), _live(s, j, _fe(j, f), 0), 0)),
            pl.BlockSpec((1, 1, FT),
                         lambda j, f, s: (
                             _live(s, j, s[j] * NF + _fe(j, f), 0), 0, 0)),
            pl.BlockSpec((1, H, FT),
                         lambda j, f, s: (_e(s, j), 0, _live(s, j, _fe(j, f), 0))),
            pl.BlockSpec((1, 1, H), lambda j, f, s: (_e(s, j), 0, 0)),
        ],
        out_specs=pl.BlockSpec((TILE, H), lambda j, f, s: (j, 0)),
    )
    return pl.pallas_call(
        _ffn_body,
        grid_spec=grid_spec,
        out_shape=jax.ShapeDtypeStruct((NPAD, H), jnp.float32),
        compiler_params=pltpu.CompilerParams(
            dimension_semantics=("arbitrary", "arbitrary")),
    )(b2e, xs, W1, b1.reshape(E * NF, 1, FT), W2, b2.reshape(E, 1, H))


@functools.partial(
    pl.kernel,
    out_type=[jax.ShapeDtypeStruct((T, H), jnp.float32),
              jax.ShapeDtypeStruct((T, H), jnp.float32)],
    mesh=_mesh,
    scratch_types=[pltpu.VMEM((16,), jnp.int32),
                   pltpu.VMEM((16,), jnp.int32),
                   pltpu.VMEM((16, H), jnp.float32),
                   pltpu.VMEM((16, H), jnp.float32),
                   pltpu.SemaphoreType.DMA,
                   pltpu.SemaphoreType.DMA],
)
def _combine(y_hbm, se_hbm, so_hbm, ya_hbm, yb_hbm,
             ie_v, io_v, ra_v, rb_v, sem_a, sem_b):
    cid = lax.axis_index("c")
    sid = lax.axis_index("s")
    wid = sid * 2 + cid
    t0 = wid * TSPAN

    def body(i, carry):
        tt = t0 + i * 16
        pltpu.sync_copy(se_hbm.at[pl.ds(tt, 16)], ie_v)
        pltpu.sync_copy(so_hbm.at[pl.ds(tt, 16)], io_v)
        ca = pltpu.async_copy(y_hbm.at[ie_v], ra_v, sem_a)
        cb = pltpu.async_copy(y_hbm.at[io_v], rb_v, sem_b)
        ca.wait()
        cb.wait()
        pltpu.sync_copy(ra_v, ya_hbm.at[pl.ds(tt, 16)])
        pltpu.sync_copy(rb_v, yb_hbm.at[pl.ds(tt, 16)])
        return carry

    lax.fori_loop(0, TSPAN // 16, body, 0)


def _add_body(a_ref, b_ref, o_ref):
    o_ref[...] = a_ref[...] + b_ref[...]


def _add(ya, yb):
    return pl.pallas_call(
        _add_body,
        grid=(T // 512,),
        in_specs=[pl.BlockSpec((512, H), lambda i: (i, 0)),
                  pl.BlockSpec((512, H), lambda i: (i, 0))],
        out_specs=pl.BlockSpec((512, H), lambda i: (i, 0)),
        out_shape=jax.ShapeDtypeStruct((T, H), jnp.float32),
    )(ya, yb)


def kernel(hidden_states, W_router, W1, b1, W2, b2):
    x2d = hidden_states.reshape(T, H)
    ep, cnts, loss11 = _router(x2d, W_router)
    e_flat = ep.reshape(NA)
    cnt_flat = cnts.reshape(NW * 16)
    slots, b2e = _dispatch(e_flat, cnt_flat)
    se = slots[0::2]
    so = slots[1::2]
    xs = _scatter_rows(x2d, se, so)
    y = _ffn(b2e, xs, W1, b1, W2, b2)
    ya, yb = _combine(y, se, so)
    out = _add(ya, yb).reshape(B, S, H)
    return (out, loss11[0, 0])


# speedup vs baseline: 1.0187x; 1.0088x over previous
import functools

import jax
import jax.numpy as jnp
from jax import lax
from jax.experimental import pallas as pl
from jax.experimental.pallas import tpu as pltpu
from jax.experimental.pallas import tpu_sc as plsc

B, S, H = 2, 2048, 2048
E, TOPK, DFF = 8, 2, 8192
AUX_COEF = 0.001
Z_COEF = 0.001

T = B * S
NA = TOPK * T
TILE = 512
LOG2_TILE = 9
NPAD = 11776
NBLK = NPAD // TILE
FT = 1024
NF = DFF // FT
RB = 128
NRB = T // RB
NW = 32
SPAN = NA // NW
TSPAN = T // NW

_mesh = plsc.VectorSubcoreMesh(core_axis_name="c", subcore_axis_name="s")


def _router_body(x_ref, wr_ref, ep_ref, cnt_ref, loss_ref, acc_ref):
    i = pl.program_id(0)
    xb = x_ref[...].astype(jnp.bfloat16)
    wr = wr_ref[...].astype(jnp.bfloat16)
    logits = lax.dot_general(xb, wr, (((1,), (1,)), ((), ())),
                             preferred_element_type=jnp.float32)
    m = jnp.max(logits, axis=1, keepdims=True)
    p = jnp.exp(logits - m)
    probs = p / jnp.sum(p, axis=1, keepdims=True)
    idx = lax.broadcasted_iota(jnp.int32, (RB, E), 1)
    p1 = jnp.max(probs, axis=1, keepdims=True)
    top1 = jnp.min(jnp.where(probs == p1, idx, E), axis=1, keepdims=True)
    pm = jnp.where(idx == top1, -jnp.inf, probs)
    p2 = jnp.max(pm, axis=1, keepdims=True)
    top2 = jnp.min(jnp.where(pm == p2, idx, E), axis=1, keepdims=True)
    ep_ref[:, 0:1] = top1
    ep_ref[:, 1:2] = top2
    idx16 = lax.broadcasted_iota(jnp.int32, (RB, 16), 1)
    cnt = (jnp.sum((idx16 == top1).astype(jnp.int32), axis=0, keepdims=True)
           + jnp.sum((idx16 == top2).astype(jnp.int32), axis=0, keepdims=True))
    cnt_ref[0] = cnt
    lse = m + jnp.log(jnp.sum(p, axis=1, keepdims=True))
    aux_p = jnp.sum(lse) - jnp.sum(logits) / E
    z_p = jnp.sum(logits * logits)

    @pl.when(i == 0)
    def _():
        acc_ref[0] = 0.0
        acc_ref[1] = 0.0

    acc_ref[0] += aux_p
    acc_ref[1] += z_p

    @pl.when(i == NRB - 1)
    def _():
        aux = (acc_ref[0] - T * jnp.log(jnp.float32(E))) / B * AUX_COEF
        z = acc_ref[1] / (T * E) * Z_COEF
        loss_ref[...] = jnp.broadcast_to(aux + z, (1, 1))


def _router(x2d, W_router):
    return pl.pallas_call(
        _router_body,
        grid=(NRB,),
        in_specs=[pl.BlockSpec((RB, H), lambda i: (i, 0)),
                  pl.BlockSpec((E, H), lambda i: (0, 0))],
        out_specs=[pl.BlockSpec((RB, TOPK), lambda i: (i, 0)),
                   pl.BlockSpec((1, 1, 16), lambda i: (i, 0, 0)),
                   pl.BlockSpec((1, 1), lambda i: (0, 0))],
        out_shape=[jax.ShapeDtypeStruct((T, TOPK), jnp.int32),
                   jax.ShapeDtypeStruct((NRB, 1, 16), jnp.int32),
                   jax.ShapeDtypeStruct((1, 1), jnp.float32)],
        scratch_shapes=[pltpu.SMEM((2,), jnp.float32)],
    )(x2d, W_router)


@functools.partial(
    pl.kernel,
    out_type=[jax.ShapeDtypeStruct((NA,), jnp.int32),
              jax.ShapeDtypeStruct((32,), jnp.int32)],
    mesh=_mesh,
    scratch_types=[pltpu.VMEM((SPAN,), jnp.int32),
                   pltpu.VMEM((SPAN,), jnp.int32),
                   pltpu.VMEM((NW * 16,), jnp.int32),
                   pltpu.VMEM((32,), jnp.int32)],
    compiler_params=pltpu.CompilerParams(needs_layout_passes=False),
)
def _dispatch(e_hbm, cnt_hbm, slots_hbm, b2e_hbm, eb_v, sl_v, allc_v, b2e_v):
    cid = lax.axis_index("c")
    sid = lax.axis_index("s")
    wid = sid * 2 + cid
    base = wid * SPAN
    pltpu.sync_copy(cnt_hbm, allc_v)
    pltpu.sync_copy(e_hbm.at[pl.ds(base, SPAN)], eb_v)
    lane = lax.iota(jnp.int32, 16)
    tot = jnp.zeros((16,), jnp.int32)
    pre = jnp.zeros((16,), jnp.int32)
    for w in range(NW):
        cw = allc_v[pl.ds(w * 16, 16)]
        tot = tot + cw
        pre = pre + jnp.where(w < wid, cw, 0)
    ptot = ((tot + (TILE - 1)) >> LOG2_TILE) << LOG2_TILE
    cums = plsc.cumsum(ptot)
    offs = cums - ptot
    run = offs + pre
    for ch in range(SPAN // 16):
        ev = eb_v[pl.ds(ch * 16, 16)]
        sl = jnp.zeros((16,), jnp.int32)
        for v in range(E):
            msk = ev == v
            incl = plsc.cumsum(jnp.where(msk, 1, 0))
            basev = jnp.sum(jnp.where(lane == v, run, 0))
            sl = jnp.where(msk, basev + incl - 1, sl)
            c = plsc.all_reduce_population_count(msk)
            run = run + jnp.where(lane == v, c, 0)
        sl_v[pl.ds(ch * 16, 16)] = sl
    pltpu.sync_copy(sl_v, slots_hbm.at[pl.ds(base, SPAN)])

    @pl.when(wid == 0)
    def _():
        offs_blk = offs >> LOG2_TILE
        total_blk = jnp.sum(jnp.where(lane == E - 1, cums, 0)) >> LOG2_TILE
        for cc in range(2):
            jv = lane + cc * 16
            acc = jnp.full((16,), -1, jnp.int32)
            for v in range(E):
                ob = jnp.sum(jnp.where(lane == v, offs_blk, 0))
                acc = acc + jnp.where(jv >= ob, 1, 0)
            b2e_v[pl.ds(cc * 16, 16)] = jnp.where(jv < total_blk, acc, -1)
        pltpu.sync_copy(b2e_v, b2e_hbm)


@functools.partial(
    pl.kernel,
    out_type=jax.ShapeDtypeStruct((NPAD, H), jnp.float32),
    mesh=_mesh,
    scratch_types=[pltpu.VMEM((16,), jnp.int32),
                   pltpu.VMEM((16,), jnp.int32),
                   pltpu.VMEM((16,), jnp.int32),
                   pltpu.VMEM((16,), jnp.int32),
                   pltpu.VMEM((16, H), jnp.float32),
                   pltpu.VMEM((16, H), jnp.float32),
                   pltpu.SemaphoreType.DMA,
                   pltpu.SemaphoreType.DMA,
                   pltpu.SemaphoreType.DMA,
                   pltpu.SemaphoreType.DMA],
)
def _scatter_rows(x_hbm, se_hbm, so_hbm, xs_hbm,
                  ie0, io0, ie1, io1, r0, r1, sa0, sb0, sa1, sb1):
    cid = lax.axis_index("c")
    sid = lax.axis_index("s")
    wid = sid * 2 + cid
    t0 = wid * TSPAN
    ie = (ie0, ie1)
    io = (io0, io1)
    rows = (r0, r1)
    sem_a = (sa0, sa1)
    sem_b = (sb0, sb1)
    pend = [None, None]
    for ch in range(TSPAN // 16):
        b = ch & 1
        if pend[b] is not None:
            pend[b][0].wait()
            pend[b][1].wait()
        tt = t0 + ch * 16
        pltpu.sync_copy(se_hbm.at[pl.ds(tt, 16)], ie[b])
        pltpu.sync_copy(so_hbm.at[pl.ds(tt, 16)], io[b])
        pltpu.sync_copy(x_hbm.at[pl.ds(tt, 16)], rows[b])
        d1 = pltpu.async_copy(rows[b], xs_hbm.at[ie[b]], sem_a[b])
        d2 = pltpu.async_copy(rows[b], xs_hbm.at[io[b]], sem_b[b])
        pend[b] = (d1, d2)
    for b in range(2):
        if pend[b] is not None:
            pend[b][0].wait()
            pend[b][1].wait()


def _ffn_body(b2e_ref, x_ref, w1_ref, b1_ref, w2_ref, b2_ref, y_ref):
    j = pl.program_id(0)
    f = pl.program_id(1)
    fe = jnp.where((j & 1) == 1, NF - 1 - f, f)
    e = b2e_ref[j]

    @pl.when(e >= 0)
    def _():
        h = lax.dot_general(x_ref[...], w1_ref[0], (((1,), (1,)), ((), ())),
                            precision=lax.Precision.DEFAULT,
                            preferred_element_type=jnp.float32)
        h = h + b1_ref[0]
        h = h * lax.logistic(h)
        yk = lax.dot_general(h, w2_ref[0], (((1,), (1,)), ((), ())),
                             precision=lax.Precision.DEFAULT,
                             preferred_element_type=jnp.float32)

        @pl.when(f == 0)
        def _():
            y_ref[...] = yk + b2_ref[0]

        @pl.when(f != 0)
        def _():
            y_ref[...] = y_ref[...] + yk


def _ffn(b2e, xs, W1, b1, W2, b2):
    def _e(s, j):
        return jnp.maximum(s[j], 0)

    def _fe(j, f):
        return jnp.where((j & 1) == 1, NF - 1 - f, f)

    def _live(s, j, v, dead):
        return jnp.where(s[j] < 0, dead, v)

    grid_spec = pltpu.PrefetchScalarGridSpec(
        num_scalar_prefetch=1,
        grid=(NBLK, NF),
        in_specs=[
            pl.BlockSpec((TILE, H), lambda j, f, s: (_live(s, j, j, 0), 0)),
            pl.BlockSpec((1, FT, H),
                         lambda j, f, s: (_e(s, j), _live(s, j, _fe(j, f), 0), 0)),
            pl.BlockSpec((1, 1, FT),
                         lambda j, f, s: (
                             _live(s, j, s[j] * NF + _fe(j, f), 0), 0, 0)),
            pl.BlockSpec((1, H, FT),
                         lambda j, f, s: (_e(s, j), 0, _live(s, j, _fe(j, f), 0))),
            pl.BlockSpec((1, 1, H), lambda j, f, s: (_e(s, j), 0, 0)),
        ],
        out_specs=pl.BlockSpec((TILE, H), lambda j, f, s: (j, 0)),
    )
    return pl.pallas_call(
        _ffn_body,
        grid_spec=grid_spec,
        out_shape=jax.ShapeDtypeStruct((NPAD, H), jnp.float32),
        compiler_params=pltpu.CompilerParams(
            dimension_semantics=("arbitrary", "arbitrary")),
    )(b2e, xs, W1, b1.reshape(E * NF, 1, FT), W2, b2.reshape(E, 1, H))


@functools.partial(
    pl.kernel,
    out_type=[jax.ShapeDtypeStruct((T, H), jnp.float32),
              jax.ShapeDtypeStruct((T, H), jnp.float32)],
    mesh=_mesh,
    scratch_types=[pltpu.VMEM((16,), jnp.int32),
                   pltpu.VMEM((16,), jnp.int32),
                   pltpu.VMEM((16, H), jnp.float32),
                   pltpu.VMEM((16, H), jnp.float32),
                   pltpu.SemaphoreType.DMA,
                   pltpu.SemaphoreType.DMA],
)
def _combine(y_hbm, se_hbm, so_hbm, ya_hbm, yb_hbm,
             ie_v, io_v, ra_v, rb_v, sem_a, sem_b):
    cid = lax.axis_index("c")
    sid = lax.axis_index("s")
    wid = sid * 2 + cid
    t0 = wid * TSPAN

    def body(i, carry):
        tt = t0 + i * 16
        pltpu.sync_copy(se_hbm.at[pl.ds(tt, 16)], ie_v)
        pltpu.sync_copy(so_hbm.at[pl.ds(tt, 16)], io_v)
        ca = pltpu.async_copy(y_hbm.at[ie_v], ra_v, sem_a)
        cb = pltpu.async_copy(y_hbm.at[io_v], rb_v, sem_b)
        ca.wait()
        cb.wait()
        pltpu.sync_copy(ra_v, ya_hbm.at[pl.ds(tt, 16)])
        pltpu.sync_copy(rb_v, yb_hbm.at[pl.ds(tt, 16)])
        return carry

    lax.fori_loop(0, TSPAN // 16, body, 0)


def _add_body(a_ref, b_ref, o_ref):
    o_ref[...] = a_ref[...] + b_ref[...]


def _add(ya, yb):
    return pl.pallas_call(
        _add_body,
        grid=(T // 512,),
        in_specs=[pl.BlockSpec((512, H), lambda i: (i, 0)),
                  pl.BlockSpec((512, H), lambda i: (i, 0))],
        out_specs=pl.BlockSpec((512, H), lambda i: (i, 0)),
        out_shape=jax.ShapeDtypeStruct((T, H), jnp.float32),
    )(ya, yb)


def kernel(hidden_states, W_router, W1, b1, W2, b2):
    x2d = hidden_states.reshape(T, H)
    ep, cnts, loss11 = _router(x2d, W_router)
    e_flat = ep.reshape(NA)
    cnt_flat = cnts.reshape(NW * 16)
    slots, b2e = _dispatch(e_flat, cnt_flat)
    se = slots[0::2]
    so = slots[1::2]
    xs = _scatter_rows(x2d, se, so)
    y = _ffn(b2e, xs, W1, b1, W2, b2)
    ya, yb = _combine(y, se, so)
    out = _add(ya, yb).reshape(B, S, H)
    return (out, loss11[0, 0])
